# baseline (speedup 1.0000x reference)
import functools

import jax
import jax.numpy as jnp
from jax import lax
from jax.experimental import pallas as pl
from jax.experimental.pallas import tpu as pltpu
from jax.experimental.pallas import tpu_sc as plsc

N = 10000
D = 128
E = 320000

NC = 2
NS = 16
NW = NC * NS

B = 128
CHUNKS = (-(-E // (NW * B)) + 3) // 4 * 4
E_PAD = NW * CHUNKS * B
GB = 4

N_PAD = 10240
ROWS_PER_TILE = N_PAD // NS
WB_STEPS = ROWS_PER_TILE // B

BM = 1024
GRID = N_PAD // BM

_PREC = jax.lax.Precision.HIGHEST


def _deg_body(dst_hbm, out_hbm, dst_v, ones_v, stage_v, deg_sh):
    c = lax.axis_index("c")
    s = lax.axis_index("s")
    wid = c * NS + s
    pltpu.sync_copy(dst_hbm.at[wid], dst_v)
    for l in range(B // 16):
        ones_v[pl.ds(l * 16, 16)] = jnp.ones((16,), jnp.float32)

    def _z(i, carry):
        stage_v[pl.ds(i * 16, 16)] = jnp.zeros((16,), jnp.float32)
        return carry
    lax.fori_loop(0, ROWS_PER_TILE // 16, _z, 0)
    base = s * ROWS_PER_TILE
    pltpu.sync_copy(stage_v, deg_sh.at[pl.ds(base, ROWS_PER_TILE)])
    plsc.subcore_barrier()

    def _chunk(j, carry):
        pltpu.sync_copy(ones_v, deg_sh.at[dst_v.at[j]], add=True)
        return carry
    lax.fori_loop(0, CHUNKS, _chunk, 0)
    plsc.subcore_barrier()

    pltpu.sync_copy(deg_sh.at[pl.ds(base, ROWS_PER_TILE)], stage_v)
    pltpu.sync_copy(stage_v, out_hbm.at[c, pl.ds(base, ROWS_PER_TILE)])


_deg_kernel = pl.kernel(
    _deg_body,
    out_type=jax.ShapeDtypeStruct((NC, N_PAD), jnp.float32),
    mesh=plsc.VectorSubcoreMesh(core_axis_name="c", subcore_axis_name="s", num_cores=NC, num_subcores=NS),
    scratch_types=[
        pltpu.VMEM((CHUNKS, B), jnp.int32),
        pltpu.VMEM((B,), jnp.float32),
        pltpu.VMEM((ROWS_PER_TILE,), jnp.float32),
        pltpu.VMEM_SHARED((N_PAD,), jnp.float32),
    ],
)


def _scatter_body(g_hbm, src_hbm, dst_hbm, out_hbm, dst_v, sring, rows_v,
                  gsem0, gsem1, isem0, isem1, isem2, isem3, acc_sh):
    c = lax.axis_index("c")
    s = lax.axis_index("s")
    wid = c * NS + s
    pltpu.sync_copy(dst_hbm.at[wid], dst_v)

    def _zrow(i, carry):
        for l in range(D // 16):
            rows_v[0, i, pl.ds(l * 16, 16)] = jnp.zeros((16,), jnp.float32)
        return carry
    lax.fori_loop(0, B, _zrow, 0)
    base = s * ROWS_PER_TILE
    for k in range(WB_STEPS):
        pltpu.sync_copy(rows_v.at[0], acc_sh.at[pl.ds(base + k * B, B)])
    plsc.subcore_barrier()

    isems = (isem0, isem1, isem2, isem3)
    gsems = (gsem0, gsem1)

    def start_i(j, r):
        pltpu.async_copy(src_hbm.at[wid, j], sring.at[r], isems[r])

    def wait_i(r):
        pltpu.make_async_copy(src_hbm.at[0, 0], sring.at[r], isems[r]).wait()

    def start_g(r, p):
        pltpu.async_copy(g_hbm.at[sring.at[r]], rows_v.at[p], gsems[p])

    def wait_g(p):
        pltpu.make_async_copy(g_hbm.at[sring.at[0]], rows_v.at[p], gsems[p]).wait()

    def scat(j, p):
        pltpu.sync_copy(rows_v.at[p], acc_sh.at[dst_v.at[j]], add=True)

    for r in range(GB):
        start_i(r, r)
    wait_i(0)
    start_g(0, 0)
    wait_i(1)
    start_g(1, 1)

    def _quad(q, carry):
        j0 = 4 * q
        wait_g(0); scat(j0, 0);     wait_i(2); start_g(2, 0); start_i(j0 + 4, 0)
        wait_g(1); scat(j0 + 1, 1); wait_i(3); start_g(3, 1); start_i(j0 + 5, 1)
        wait_g(0); scat(j0 + 2, 0); wait_i(0); start_g(0, 0); start_i(j0 + 6, 2)
        wait_g(1); scat(j0 + 3, 1); wait_i(1); start_g(1, 1); start_i(j0 + 7, 3)
        return carry
    lax.fori_loop(0, CHUNKS // 4 - 1, _quad, 0)
    j0 = CHUNKS - 4
    wait_g(0); scat(j0, 0);     wait_i(2); start_g(2, 0)
    wait_g(1); scat(j0 + 1, 1); wait_i(3); start_g(3, 1)
    wait_g(0); scat(j0 + 2, 0)
    wait_g(1); scat(j0 + 3, 1)
    plsc.subcore_barrier()

    def _wb(k, carry):
        pltpu.sync_copy(acc_sh.at[pl.ds(base + k * B, B)], rows_v.at[0])
        pltpu.sync_copy(rows_v.at[0], out_hbm.at[c, pl.ds(base + k * B, B)])
        return carry
    lax.fori_loop(0, WB_STEPS, _wb, 0)


_scatter_kernel = pl.kernel(
    _scatter_body,
    out_type=jax.ShapeDtypeStruct((NC, N_PAD, D), jnp.float32),
    mesh=plsc.VectorSubcoreMesh(core_axis_name="c", subcore_axis_name="s", num_cores=NC, num_subcores=NS),
    scratch_types=[
        pltpu.VMEM((CHUNKS, B), jnp.int32),
        pltpu.VMEM((GB, B), jnp.int32),
        pltpu.VMEM((2, B, D), jnp.float32),
        pltpu.SemaphoreType.DMA,
        pltpu.SemaphoreType.DMA,
        pltpu.SemaphoreType.DMA,
        pltpu.SemaphoreType.DMA,
        pltpu.SemaphoreType.DMA,
        pltpu.SemaphoreType.DMA,
        pltpu.VMEM_SHARED((N_PAD, D), jnp.float32),
    ],
)


def _dinv(deg_ref):
    return lax.rsqrt(deg_ref[0] + deg_ref[1] + 1.0)


def _a_body(deg_ref, x_ref, w1_ref, we_ref, be_ref, g1_ref, res_ref):
    dinv = _dinv(deg_ref)
    h1 = jnp.dot(x_ref[...], w1_ref[...], precision=_PREC,
                 preferred_element_type=jnp.float32)
    g1_ref[...] = h1 * dinv
    res_ref[...] = jnp.dot(x_ref[...], we_ref[...], precision=_PREC,
                           preferred_element_type=jnp.float32) + be_ref[...]


def _b_body(deg_ref, acc_ref, g1_ref, w2_ref, b1_ref, g2_ref):
    dinv = _dinv(deg_ref)
    acc = acc_ref[0] + acc_ref[1] + g1_ref[...]
    z = jnp.maximum(acc * dinv + b1_ref[...], 0.0)
    g2_ref[...] = jnp.dot(z, w2_ref[...], precision=_PREC,
                          preferred_element_type=jnp.float32) * dinv


def _c_body(deg_ref, acc_ref, g2_ref, res_ref, b2_ref, out_ref):
    dinv = _dinv(deg_ref)
    out_ref[...] = ((acc_ref[0] + acc_ref[1] + g2_ref[...]) * dinv
                    + b2_ref[...] + res_ref[...])


_deg_spec = pl.BlockSpec((2, BM, 1), lambda i: (0, i, 0))
_row_spec = pl.BlockSpec((BM, D), lambda i: (i, 0))
_acc_spec = pl.BlockSpec((2, BM, D), lambda i: (0, i, 0))
_w_spec = pl.BlockSpec((D, D), lambda i: (0, 0))
_b_spec = pl.BlockSpec((1, D), lambda i: (0, 0))

_stage_a = pl.pallas_call(
    _a_body,
    grid=(GRID,),
    in_specs=[_deg_spec, _row_spec, _w_spec, _w_spec, _b_spec],
    out_specs=[_row_spec, _row_spec],
    out_shape=[jax.ShapeDtypeStruct((N_PAD, D), jnp.float32)] * 2,
)

_stage_b = pl.pallas_call(
    _b_body,
    grid=(GRID,),
    in_specs=[_deg_spec, _acc_spec, _row_spec, _w_spec, _b_spec],
    out_specs=_row_spec,
    out_shape=jax.ShapeDtypeStruct((N_PAD, D), jnp.float32),
)

_stage_c = pl.pallas_call(
    _c_body,
    grid=(GRID,),
    in_specs=[_deg_spec, _acc_spec, _row_spec, _row_spec, _b_spec],
    out_specs=_row_spec,
    out_shape=jax.ShapeDtypeStruct((N_PAD, D), jnp.float32),
)


@jax.jit
def kernel(x, edge_index, W1, b1, W2, b2, We, be):
    src = edge_index[0]
    dst = edge_index[1]
    pad = E_PAD - E
    src3 = jnp.concatenate([src, jnp.zeros((pad,), jnp.int32)]).reshape(NW, CHUNKS, B)
    dst3 = jnp.concatenate([dst, jnp.full((pad,), N, jnp.int32)]).reshape(NW, CHUNKS, B)
    x_pad = jnp.concatenate([x, jnp.zeros((N_PAD - N, D), jnp.float32)])

    deg = _deg_kernel(dst3).reshape(NC, N_PAD, 1)
    g1, res = _stage_a(deg, x_pad, W1, We, be.reshape(1, D))
    acc1 = _scatter_kernel(g1, src3, dst3)
    g2 = _stage_b(deg, acc1, g1, W2, b1.reshape(1, D))
    acc2 = _scatter_kernel(g2, src3, dst3)
    out = _stage_c(deg, acc2, g2, res, b2.reshape(1, D))
    return out[:N]


# speedup vs baseline: 3.3761x; 3.3761x over previous
import functools

import jax
import jax.numpy as jnp
from jax import lax
from jax.experimental import pallas as pl
from jax.experimental.pallas import tpu as pltpu
from jax.experimental.pallas import tpu_sc as plsc

N = 10000
D = 128
E = 320000

NC = 2
NS = 16
NW = NC * NS

B = 128
CHUNKS = (-(-E // (NW * B)) + 3) // 4 * 4
E_PAD = NW * CHUNKS * B
GB = 4

N_PAD = 10240
ROWS_PER_TILE = N_PAD // NS
WB_STEPS = ROWS_PER_TILE // B

BM = 1024
GRID = N_PAD // BM

_PREC = jax.lax.Precision.HIGHEST


def _deg_body(dst_hbm, out_hbm, dst_v, ones_v, stage_v, deg_sh):
    c = lax.axis_index("c")
    s = lax.axis_index("s")
    wid = c * NS + s
    pltpu.sync_copy(dst_hbm.at[wid], dst_v)
    for l in range(B // 16):
        ones_v[pl.ds(l * 16, 16)] = jnp.ones((16,), jnp.float32)

    def _z(i, carry):
        stage_v[pl.ds(i * 16, 16)] = jnp.zeros((16,), jnp.float32)
        return carry
    lax.fori_loop(0, ROWS_PER_TILE // 16, _z, 0)
    base = s * ROWS_PER_TILE
    pltpu.sync_copy(stage_v, deg_sh.at[pl.ds(base, ROWS_PER_TILE)])
    plsc.subcore_barrier()

    def _chunk(j, carry):
        pltpu.sync_copy(ones_v, deg_sh.at[dst_v.at[j]], add=True)
        return carry
    lax.fori_loop(0, CHUNKS, _chunk, 0)
    plsc.subcore_barrier()

    pltpu.sync_copy(deg_sh.at[pl.ds(base, ROWS_PER_TILE)], stage_v)
    pltpu.sync_copy(stage_v, out_hbm.at[c, pl.ds(base, ROWS_PER_TILE)])


_deg_kernel = pl.kernel(
    _deg_body,
    out_type=jax.ShapeDtypeStruct((NC, N_PAD), jnp.float32),
    mesh=plsc.VectorSubcoreMesh(core_axis_name="c", subcore_axis_name="s", num_cores=NC, num_subcores=NS),
    scratch_types=[
        pltpu.VMEM((CHUNKS, B), jnp.int32),
        pltpu.VMEM((B,), jnp.float32),
        pltpu.VMEM((ROWS_PER_TILE,), jnp.float32),
        pltpu.VMEM_SHARED((N_PAD,), jnp.float32),
    ],
)


def _scatter_body(g_hbm, src_hbm, dst_hbm, out_hbm, dst_v, sring, rows_v,
                  gsem0, gsem1, isem0, isem1, isem2, isem3, acc_sh):
    c = lax.axis_index("c")
    s = lax.axis_index("s")
    wid = c * NS + s
    pltpu.sync_copy(dst_hbm.at[wid], dst_v)

    def _zrow(i, carry):
        for l in range(D // 16):
            rows_v[0, i, pl.ds(l * 16, 16)] = jnp.zeros((16,), jnp.float32)
        return carry
    lax.fori_loop(0, B, _zrow, 0)
    base = s * ROWS_PER_TILE
    for k in range(WB_STEPS):
        pltpu.sync_copy(rows_v.at[0], acc_sh.at[pl.ds(base + k * B, B)])
    plsc.subcore_barrier()

    isems = (isem0, isem1, isem2, isem3)
    gsems = (gsem0, gsem1)

    def start_i(j, r):
        pltpu.async_copy(src_hbm.at[wid, j], sring.at[r], isems[r])

    def wait_i(r):
        pltpu.make_async_copy(src_hbm.at[0, 0], sring.at[r], isems[r]).wait()

    def start_g(r, p):
        pltpu.async_copy(g_hbm.at[sring.at[r]], rows_v.at[p], gsems[p])

    def wait_g(p):
        pltpu.make_async_copy(g_hbm.at[sring.at[0]], rows_v.at[p], gsems[p]).wait()

    def scat(j, p):
        pltpu.sync_copy(rows_v.at[p], acc_sh.at[dst_v.at[j]], add=True)

    for r in range(GB):
        start_i(r, r)
    wait_i(0)
    start_g(0, 0)
    wait_i(1)
    start_g(1, 1)

    def _quad(q, carry):
        j0 = 4 * q
        wait_g(0); scat(j0, 0);     wait_i(2); start_g(2, 0); start_i(j0 + 4, 0)
        wait_g(1); scat(j0 + 1, 1); wait_i(3); start_g(3, 1); start_i(j0 + 5, 1)
        wait_g(0); scat(j0 + 2, 0); wait_i(0); start_g(0, 0); start_i(j0 + 6, 2)
        wait_g(1); scat(j0 + 3, 1); wait_i(1); start_g(1, 1); start_i(j0 + 7, 3)
        return carry
    lax.fori_loop(0, CHUNKS // 4 - 1, _quad, 0)
    j0 = CHUNKS - 4
    wait_g(0); scat(j0, 0);     wait_i(2); start_g(2, 0)
    wait_g(1); scat(j0 + 1, 1); wait_i(3); start_g(3, 1)
    wait_g(0); scat(j0 + 2, 0)
    wait_g(1); scat(j0 + 3, 1)
    plsc.subcore_barrier()

    def _wb(k, carry):
        pltpu.sync_copy(acc_sh.at[pl.ds(base + k * B, B)], rows_v.at[0])
        pltpu.sync_copy(rows_v.at[0], out_hbm.at[c, pl.ds(base + k * B, B)])
        return carry
    lax.fori_loop(0, WB_STEPS, _wb, 0)


_scatter_kernel = pl.kernel(
    _scatter_body,
    out_type=jax.ShapeDtypeStruct((NC, N_PAD, D), jnp.float32),
    mesh=plsc.VectorSubcoreMesh(core_axis_name="c", subcore_axis_name="s", num_cores=NC, num_subcores=NS),
    scratch_types=[
        pltpu.VMEM((CHUNKS, B), jnp.int32),
        pltpu.VMEM((GB, B), jnp.int32),
        pltpu.VMEM((2, B, D), jnp.float32),
        pltpu.SemaphoreType.DMA,
        pltpu.SemaphoreType.DMA,
        pltpu.SemaphoreType.DMA,
        pltpu.SemaphoreType.DMA,
        pltpu.SemaphoreType.DMA,
        pltpu.SemaphoreType.DMA,
        pltpu.VMEM_SHARED((N_PAD, D), jnp.float32),
    ],
)


def _dinv(deg_ref):
    return lax.rsqrt(deg_ref[0] + deg_ref[1] + 1.0)


def _a_body(deg_ref, x_ref, w1_ref, we_ref, be_ref, g1_ref, res_ref):
    dinv = _dinv(deg_ref)
    h1 = jnp.dot(x_ref[...], w1_ref[...], precision=_PREC,
                 preferred_element_type=jnp.float32)
    g1_ref[...] = h1 * dinv
    res_ref[...] = jnp.dot(x_ref[...], we_ref[...], precision=_PREC,
                           preferred_element_type=jnp.float32) + be_ref[...]


def _b_body(deg_ref, acc_ref, g1_ref, w2_ref, b1_ref, g2_ref):
    dinv = _dinv(deg_ref)
    acc = acc_ref[0] + acc_ref[1] + g1_ref[...]
    z = jnp.maximum(acc * dinv + b1_ref[...], 0.0)
    g2_ref[...] = jnp.dot(z, w2_ref[...], precision=_PREC,
                          preferred_element_type=jnp.float32) * dinv


def _c_body(deg_ref, acc_ref, g2_ref, res_ref, b2_ref, out_ref):
    dinv = _dinv(deg_ref)
    out_ref[...] = ((acc_ref[0] + acc_ref[1] + g2_ref[...]) * dinv
                    + b2_ref[...] + res_ref[...])


_deg_spec = pl.BlockSpec((2, BM, 1), lambda i: (0, i, 0))
_row_spec = pl.BlockSpec((BM, D), lambda i: (i, 0))
_acc_spec = pl.BlockSpec((2, BM, D), lambda i: (0, i, 0))
_w_spec = pl.BlockSpec((D, D), lambda i: (0, 0))
_b_spec = pl.BlockSpec((1, D), lambda i: (0, 0))

_stage_a = pl.pallas_call(
    _a_body,
    grid=(GRID,),
    in_specs=[_deg_spec, _row_spec, _w_spec, _w_spec, _b_spec],
    out_specs=[_row_spec, _row_spec],
    out_shape=[jax.ShapeDtypeStruct((N_PAD, D), jnp.float32)] * 2,
)

_stage_b = pl.pallas_call(
    _b_body,
    grid=(GRID,),
    in_specs=[_deg_spec, _acc_spec, _row_spec, _w_spec, _b_spec],
    out_specs=_row_spec,
    out_shape=jax.ShapeDtypeStruct((N_PAD, D), jnp.float32),
)

_stage_c = pl.pallas_call(
    _c_body,
    grid=(GRID,),
    in_specs=[_deg_spec, _acc_spec, _row_spec, _row_spec, _b_spec],
    out_specs=_row_spec,
    out_shape=jax.ShapeDtypeStruct((N_PAD, D), jnp.float32),
)


@jax.jit
def kernel(x, edge_index, W1, b1, W2, b2, We, be):
    src = edge_index[0]
    dst = edge_index[1]
    pad = E_PAD - E
    pad_iota = jnp.arange(pad, dtype=jnp.int32)
    src3 = jnp.concatenate([src, pad_iota % N]).reshape(NW, CHUNKS, B)
    dst3 = jnp.concatenate([dst, N + pad_iota % (N_PAD - N)]).reshape(NW, CHUNKS, B)
    x_pad = jnp.concatenate([x, jnp.zeros((N_PAD - N, D), jnp.float32)])

    deg = _deg_kernel(dst3).reshape(NC, N_PAD, 1)
    g1, res = _stage_a(deg, x_pad, W1, We, be.reshape(1, D))
    acc1 = _scatter_kernel(g1, src3, dst3)
    g2 = _stage_b(deg, acc1, g1, W2, b1.reshape(1, D))
    acc2 = _scatter_kernel(g2, src3, dst3)
    out = _stage_c(deg, acc2, g2, res, b2.reshape(1, D))
    return out[:N]


# speedup vs baseline: 3.3890x; 1.0038x over previous
import functools

import jax
import jax.numpy as jnp
from jax import lax
from jax.experimental import pallas as pl
from jax.experimental.pallas import tpu as pltpu
from jax.experimental.pallas import tpu_sc as plsc

N = 10000
D = 128
E = 320000

NC = 2
NS = 16
NW = NC * NS

B = 128
CHUNKS = (-(-E // (NW * B)) + 3) // 4 * 4
E_PAD = NW * CHUNKS * B
GB = 4

N_PAD = 10240
ROWS_PER_TILE = N_PAD // NS
WB_STEPS = ROWS_PER_TILE // B

BM = 1000
GRID = N // BM

_PREC = jax.lax.Precision.HIGHEST


def _deg_body(dst_hbm, out_hbm, dst_v, ones_v, stage_v, deg_sh):
    c = lax.axis_index("c")
    s = lax.axis_index("s")
    wid = c * NS + s
    pltpu.sync_copy(dst_hbm.at[wid], dst_v)
    for l in range(B // 16):
        ones_v[pl.ds(l * 16, 16)] = jnp.ones((16,), jnp.float32)

    def _z(i, carry):
        stage_v[pl.ds(i * 16, 16)] = jnp.zeros((16,), jnp.float32)
        return carry
    lax.fori_loop(0, ROWS_PER_TILE // 16, _z, 0)
    base = s * ROWS_PER_TILE
    pltpu.sync_copy(stage_v, deg_sh.at[pl.ds(base, ROWS_PER_TILE)])
    plsc.subcore_barrier()

    def _chunk(j, carry):
        pltpu.sync_copy(ones_v, deg_sh.at[dst_v.at[j]], add=True)
        return carry
    lax.fori_loop(0, CHUNKS, _chunk, 0)
    plsc.subcore_barrier()

    pltpu.sync_copy(deg_sh.at[pl.ds(base, ROWS_PER_TILE)], stage_v)
    pltpu.sync_copy(stage_v, out_hbm.at[c, pl.ds(base, ROWS_PER_TILE)])


_deg_kernel = pl.kernel(
    _deg_body,
    out_type=jax.ShapeDtypeStruct((NC, N_PAD), jnp.float32),
    mesh=plsc.VectorSubcoreMesh(core_axis_name="c", subcore_axis_name="s", num_cores=NC, num_subcores=NS),
    scratch_types=[
        pltpu.VMEM((CHUNKS, B), jnp.int32),
        pltpu.VMEM((B,), jnp.float32),
        pltpu.VMEM((ROWS_PER_TILE,), jnp.float32),
        pltpu.VMEM_SHARED((N_PAD,), jnp.float32),
    ],
)


def _scatter_body(g_hbm, src_hbm, dst_hbm, out_hbm, dst_v, sring, rows_v,
                  gsem0, gsem1, isem0, isem1, isem2, isem3, acc_sh):
    c = lax.axis_index("c")
    s = lax.axis_index("s")
    wid = c * NS + s
    pltpu.sync_copy(dst_hbm.at[wid], dst_v)

    def _zrow(i, carry):
        for l in range(D // 16):
            rows_v[0, i, pl.ds(l * 16, 16)] = jnp.zeros((16,), jnp.float32)
        return carry
    lax.fori_loop(0, B, _zrow, 0)
    base = s * ROWS_PER_TILE
    for k in range(WB_STEPS):
        pltpu.sync_copy(rows_v.at[0], acc_sh.at[pl.ds(base + k * B, B)])
    plsc.subcore_barrier()

    isems = (isem0, isem1, isem2, isem3)
    gsems = (gsem0, gsem1)

    def start_i(j, r):
        pltpu.async_copy(src_hbm.at[wid, j], sring.at[r], isems[r])

    def wait_i(r):
        pltpu.make_async_copy(src_hbm.at[0, 0], sring.at[r], isems[r]).wait()

    def start_g(r, p):
        pltpu.async_copy(g_hbm.at[sring.at[r]], rows_v.at[p], gsems[p])

    def wait_g(p):
        pltpu.make_async_copy(g_hbm.at[sring.at[0]], rows_v.at[p], gsems[p]).wait()

    def scat(j, p):
        pltpu.sync_copy(rows_v.at[p], acc_sh.at[dst_v.at[j]], add=True)

    for r in range(GB):
        start_i(r, r)
    wait_i(0)
    start_g(0, 0)
    wait_i(1)
    start_g(1, 1)

    def _quad(q, carry):
        j0 = 4 * q
        wait_g(0); scat(j0, 0);     wait_i(2); start_g(2, 0); start_i(j0 + 4, 0)
        wait_g(1); scat(j0 + 1, 1); wait_i(3); start_g(3, 1); start_i(j0 + 5, 1)
        wait_g(0); scat(j0 + 2, 0); wait_i(0); start_g(0, 0); start_i(j0 + 6, 2)
        wait_g(1); scat(j0 + 3, 1); wait_i(1); start_g(1, 1); start_i(j0 + 7, 3)
        return carry
    lax.fori_loop(0, CHUNKS // 4 - 1, _quad, 0)
    j0 = CHUNKS - 4
    wait_g(0); scat(j0, 0);     wait_i(2); start_g(2, 0)
    wait_g(1); scat(j0 + 1, 1); wait_i(3); start_g(3, 1)
    wait_g(0); scat(j0 + 2, 0)
    wait_g(1); scat(j0 + 3, 1)
    plsc.subcore_barrier()

    def _wb(k, carry):
        pltpu.sync_copy(acc_sh.at[pl.ds(base + k * B, B)], rows_v.at[0])
        pltpu.sync_copy(rows_v.at[0], out_hbm.at[c, pl.ds(base + k * B, B)])
        return carry
    lax.fori_loop(0, WB_STEPS, _wb, 0)


_scatter_kernel = pl.kernel(
    _scatter_body,
    out_type=jax.ShapeDtypeStruct((NC, N_PAD, D), jnp.float32),
    mesh=plsc.VectorSubcoreMesh(core_axis_name="c", subcore_axis_name="s", num_cores=NC, num_subcores=NS),
    scratch_types=[
        pltpu.VMEM((CHUNKS, B), jnp.int32),
        pltpu.VMEM((GB, B), jnp.int32),
        pltpu.VMEM((2, B, D), jnp.float32),
        pltpu.SemaphoreType.DMA,
        pltpu.SemaphoreType.DMA,
        pltpu.SemaphoreType.DMA,
        pltpu.SemaphoreType.DMA,
        pltpu.SemaphoreType.DMA,
        pltpu.SemaphoreType.DMA,
        pltpu.VMEM_SHARED((N_PAD, D), jnp.float32),
    ],
)


def _dinv(deg_ref):
    return lax.rsqrt(deg_ref[0] + deg_ref[1] + 1.0)


def _a_body(deg_ref, x_ref, w1_ref, we_ref, be_ref, g1_ref, res_ref):
    dinv = _dinv(deg_ref)
    h1 = jnp.dot(x_ref[...], w1_ref[...], precision=_PREC,
                 preferred_element_type=jnp.float32)
    g1_ref[...] = h1 * dinv
    res_ref[...] = jnp.dot(x_ref[...], we_ref[...], precision=_PREC,
                           preferred_element_type=jnp.float32) + be_ref[...]


def _b_body(deg_ref, acc_ref, g1_ref, w2_ref, b1_ref, g2_ref):
    dinv = _dinv(deg_ref)
    acc = acc_ref[0] + acc_ref[1] + g1_ref[...]
    z = jnp.maximum(acc * dinv + b1_ref[...], 0.0)
    g2_ref[...] = jnp.dot(z, w2_ref[...], precision=_PREC,
                          preferred_element_type=jnp.float32) * dinv


def _c_body(deg_ref, acc_ref, g2_ref, res_ref, b2_ref, out_ref):
    dinv = _dinv(deg_ref)
    out_ref[...] = ((acc_ref[0] + acc_ref[1] + g2_ref[...]) * dinv
                    + b2_ref[...] + res_ref[...])


_deg_spec = pl.BlockSpec((2, BM, 1), lambda i: (0, i, 0))
_row_spec = pl.BlockSpec((BM, D), lambda i: (i, 0))
_acc_spec = pl.BlockSpec((2, BM, D), lambda i: (0, i, 0))
_w_spec = pl.BlockSpec((D, D), lambda i: (0, 0))
_b_spec = pl.BlockSpec((1, D), lambda i: (0, 0))

_stage_a = pl.pallas_call(
    _a_body,
    grid=(GRID,),
    in_specs=[_deg_spec, _row_spec, _w_spec, _w_spec, _b_spec],
    out_specs=[_row_spec, _row_spec],
    out_shape=[jax.ShapeDtypeStruct((N, D), jnp.float32)] * 2,
)

_stage_b = pl.pallas_call(
    _b_body,
    grid=(GRID,),
    in_specs=[_deg_spec, _acc_spec, _row_spec, _w_spec, _b_spec],
    out_specs=_row_spec,
    out_shape=jax.ShapeDtypeStruct((N, D), jnp.float32),
)

_stage_c = pl.pallas_call(
    _c_body,
    grid=(GRID,),
    in_specs=[_deg_spec, _acc_spec, _row_spec, _row_spec, _b_spec],
    out_specs=_row_spec,
    out_shape=jax.ShapeDtypeStruct((N, D), jnp.float32),
)


@jax.jit
def kernel(x, edge_index, W1, b1, W2, b2, We, be):
    src = edge_index[0]
    dst = edge_index[1]
    pad = E_PAD - E
    pad_iota = jnp.arange(pad, dtype=jnp.int32)
    src3 = jnp.concatenate([src, pad_iota % N]).reshape(NW, CHUNKS, B)
    dst3 = jnp.concatenate([dst, N + pad_iota % (N_PAD - N)]).reshape(NW, CHUNKS, B)
    deg = _deg_kernel(dst3).reshape(NC, N_PAD, 1)
    g1, res = _stage_a(deg, x, W1, We, be.reshape(1, D))
    acc1 = _scatter_kernel(g1, src3, dst3)
    g2 = _stage_b(deg, acc1, g1, W2, b1.reshape(1, D))
    acc2 = _scatter_kernel(g2, src3, dst3)
    return _stage_c(deg, acc2, g2, res, b2.reshape(1, D))


# speedup vs baseline: 3.5024x; 1.0335x over previous
import functools

import jax
import jax.numpy as jnp
from jax import lax
from jax.experimental import pallas as pl
from jax.experimental.pallas import tpu as pltpu
from jax.experimental.pallas import tpu_sc as plsc

N = 10000
D = 128
E = 320000

NC = 2
NS = 16
NW = NC * NS

B = 128
CHUNKS = (-(-E // (NW * B)) + 3) // 4 * 4
E_PAD = NW * CHUNKS * B
GB = 4

N_PAD = 10240
ROWS_PER_TILE = N_PAD // NS
WB_STEPS = ROWS_PER_TILE // B

BM = 1000
GRID = N // BM

_PREC = jax.lax.Precision.DEFAULT


def _deg_body(dst_hbm, out_hbm, dst_v, ones_v, stage_v, deg_sh):
    c = lax.axis_index("c")
    s = lax.axis_index("s")
    wid = c * NS + s
    pltpu.sync_copy(dst_hbm.at[wid], dst_v)
    for l in range(B // 16):
        ones_v[pl.ds(l * 16, 16)] = jnp.ones((16,), jnp.float32)

    def _z(i, carry):
        stage_v[pl.ds(i * 16, 16)] = jnp.zeros((16,), jnp.float32)
        return carry
    lax.fori_loop(0, ROWS_PER_TILE // 16, _z, 0)
    base = s * ROWS_PER_TILE
    pltpu.sync_copy(stage_v, deg_sh.at[pl.ds(base, ROWS_PER_TILE)])
    plsc.subcore_barrier()

    def _chunk(j, carry):
        pltpu.sync_copy(ones_v, deg_sh.at[dst_v.at[j]], add=True)
        return carry
    lax.fori_loop(0, CHUNKS, _chunk, 0)
    plsc.subcore_barrier()

    pltpu.sync_copy(deg_sh.at[pl.ds(base, ROWS_PER_TILE)], stage_v)
    pltpu.sync_copy(stage_v, out_hbm.at[c, pl.ds(base, ROWS_PER_TILE)])


_deg_kernel = pl.kernel(
    _deg_body,
    out_type=jax.ShapeDtypeStruct((NC, N_PAD), jnp.float32),
    mesh=plsc.VectorSubcoreMesh(core_axis_name="c", subcore_axis_name="s", num_cores=NC, num_subcores=NS),
    scratch_types=[
        pltpu.VMEM((CHUNKS, B), jnp.int32),
        pltpu.VMEM((B,), jnp.float32),
        pltpu.VMEM((ROWS_PER_TILE,), jnp.float32),
        pltpu.VMEM_SHARED((N_PAD,), jnp.float32),
    ],
)


def _scatter_body(g_hbm, src_hbm, dst_hbm, out_hbm, dst_v, sring, rows_v,
                  gsem0, gsem1, isem0, isem1, isem2, isem3, acc_sh):
    c = lax.axis_index("c")
    s = lax.axis_index("s")
    wid = c * NS + s
    pltpu.sync_copy(dst_hbm.at[wid], dst_v)

    def _zrow(i, carry):
        for l in range(D // 16):
            rows_v[0, i, pl.ds(l * 16, 16)] = jnp.zeros((16,), jnp.float32)
        return carry
    lax.fori_loop(0, B, _zrow, 0)
    base = s * ROWS_PER_TILE
    for k in range(WB_STEPS):
        pltpu.sync_copy(rows_v.at[0], acc_sh.at[pl.ds(base + k * B, B)])
    plsc.subcore_barrier()

    isems = (isem0, isem1, isem2, isem3)
    gsems = (gsem0, gsem1)

    def start_i(j, r):
        pltpu.async_copy(src_hbm.at[wid, j], sring.at[r], isems[r])

    def wait_i(r):
        pltpu.make_async_copy(src_hbm.at[0, 0], sring.at[r], isems[r]).wait()

    def start_g(r, p):
        pltpu.async_copy(g_hbm.at[sring.at[r]], rows_v.at[p], gsems[p])

    def wait_g(p):
        pltpu.make_async_copy(g_hbm.at[sring.at[0]], rows_v.at[p], gsems[p]).wait()

    def scat(j, p):
        pltpu.sync_copy(rows_v.at[p], acc_sh.at[dst_v.at[j]], add=True)

    for r in range(GB):
        start_i(r, r)
    wait_i(0)
    start_g(0, 0)
    wait_i(1)
    start_g(1, 1)

    def _quad(q, carry):
        j0 = 4 * q
        wait_g(0); scat(j0, 0);     wait_i(2); start_g(2, 0); start_i(j0 + 4, 0)
        wait_g(1); scat(j0 + 1, 1); wait_i(3); start_g(3, 1); start_i(j0 + 5, 1)
        wait_g(0); scat(j0 + 2, 0); wait_i(0); start_g(0, 0); start_i(j0 + 6, 2)
        wait_g(1); scat(j0 + 3, 1); wait_i(1); start_g(1, 1); start_i(j0 + 7, 3)
        return carry
    lax.fori_loop(0, CHUNKS // 4 - 1, _quad, 0)
    j0 = CHUNKS - 4
    wait_g(0); scat(j0, 0);     wait_i(2); start_g(2, 0)
    wait_g(1); scat(j0 + 1, 1); wait_i(3); start_g(3, 1)
    wait_g(0); scat(j0 + 2, 0)
    wait_g(1); scat(j0 + 3, 1)
    plsc.subcore_barrier()

    def _wb(k, carry):
        pltpu.sync_copy(acc_sh.at[pl.ds(base + k * B, B)], rows_v.at[0])
        pltpu.sync_copy(rows_v.at[0], out_hbm.at[c, pl.ds(base + k * B, B)])
        return carry
    lax.fori_loop(0, WB_STEPS, _wb, 0)


_scatter_kernel = pl.kernel(
    _scatter_body,
    out_type=jax.ShapeDtypeStruct((NC, N_PAD, D), jnp.float32),
    mesh=plsc.VectorSubcoreMesh(core_axis_name="c", subcore_axis_name="s", num_cores=NC, num_subcores=NS),
    scratch_types=[
        pltpu.VMEM((CHUNKS, B), jnp.int32),
        pltpu.VMEM((GB, B), jnp.int32),
        pltpu.VMEM((2, B, D), jnp.float32),
        pltpu.SemaphoreType.DMA,
        pltpu.SemaphoreType.DMA,
        pltpu.SemaphoreType.DMA,
        pltpu.SemaphoreType.DMA,
        pltpu.SemaphoreType.DMA,
        pltpu.SemaphoreType.DMA,
        pltpu.VMEM_SHARED((N_PAD, D), jnp.float32),
    ],
)


def _dinv(deg_ref):
    return lax.rsqrt(deg_ref[0] + deg_ref[1] + 1.0)


def _a_body(deg_ref, x_ref, w1_ref, we_ref, be_ref, g1_ref, res_ref):
    dinv = _dinv(deg_ref)
    h1 = jnp.dot(x_ref[...], w1_ref[...], precision=_PREC,
                 preferred_element_type=jnp.float32)
    g1_ref[...] = h1 * dinv
    res_ref[...] = jnp.dot(x_ref[...], we_ref[...], precision=_PREC,
                           preferred_element_type=jnp.float32) + be_ref[...]


def _b_body(deg_ref, acc_ref, g1_ref, w2_ref, b1_ref, g2_ref):
    dinv = _dinv(deg_ref)
    acc = acc_ref[0] + acc_ref[1] + g1_ref[...]
    z = jnp.maximum(acc * dinv + b1_ref[...], 0.0)
    g2_ref[...] = jnp.dot(z, w2_ref[...], precision=_PREC,
                          preferred_element_type=jnp.float32) * dinv


def _c_body(deg_ref, acc_ref, g2_ref, res_ref, b2_ref, out_ref):
    dinv = _dinv(deg_ref)
    out_ref[...] = ((acc_ref[0] + acc_ref[1] + g2_ref[...]) * dinv
                    + b2_ref[...] + res_ref[...])


_deg_spec = pl.BlockSpec((2, BM, 1), lambda i: (0, i, 0))
_row_spec = pl.BlockSpec((BM, D), lambda i: (i, 0))
_acc_spec = pl.BlockSpec((2, BM, D), lambda i: (0, i, 0))
_w_spec = pl.BlockSpec((D, D), lambda i: (0, 0))
_b_spec = pl.BlockSpec((1, D), lambda i: (0, 0))

_stage_a = pl.pallas_call(
    _a_body,
    grid=(GRID,),
    in_specs=[_deg_spec, _row_spec, _w_spec, _w_spec, _b_spec],
    out_specs=[_row_spec, _row_spec],
    out_shape=[jax.ShapeDtypeStruct((N, D), jnp.float32)] * 2,
)

_stage_b = pl.pallas_call(
    _b_body,
    grid=(GRID,),
    in_specs=[_deg_spec, _acc_spec, _row_spec, _w_spec, _b_spec],
    out_specs=_row_spec,
    out_shape=jax.ShapeDtypeStruct((N, D), jnp.float32),
)

_stage_c = pl.pallas_call(
    _c_body,
    grid=(GRID,),
    in_specs=[_deg_spec, _acc_spec, _row_spec, _row_spec, _b_spec],
    out_specs=_row_spec,
    out_shape=jax.ShapeDtypeStruct((N, D), jnp.float32),
)


@jax.jit
def kernel(x, edge_index, W1, b1, W2, b2, We, be):
    src = edge_index[0]
    dst = edge_index[1]
    pad = E_PAD - E
    pad_iota = jnp.arange(pad, dtype=jnp.int32)
    src3 = jnp.concatenate([src, pad_iota % N]).reshape(NW, CHUNKS, B)
    dst3 = jnp.concatenate([dst, N + pad_iota % (N_PAD - N)]).reshape(NW, CHUNKS, B)
    deg = _deg_kernel(dst3).reshape(NC, N_PAD, 1)
    g1, res = _stage_a(deg, x, W1, We, be.reshape(1, D))
    acc1 = _scatter_kernel(g1, src3, dst3)
    g2 = _stage_b(deg, acc1, g1, W2, b1.reshape(1, D))
    acc2 = _scatter_kernel(g2, src3, dst3)
    return _stage_c(deg, acc2, g2, res, b2.reshape(1, D))
